# initial kernel scaffold (unmeasured)
import jax
import jax.numpy as jnp
from jax import lax
from jax.experimental import pallas as pl
from jax.experimental.pallas import tpu as pltpu

N_DEV = 4


def kernel(x, w_mat):
    m_total, k_shard = x.shape
    k_total, n = w_mat.shape
    m_per = m_total // N_DEV

    def body(x_ref, w_ref, out_ref, comm_ref, send_sems, recv_sems):
        my = lax.axis_index("i")

        barrier_sem = pltpu.get_barrier_semaphore()
        for d in range(1, N_DEV):
            peer = lax.rem(my + d, N_DEV)
            pl.semaphore_signal(
                barrier_sem, inc=1,
                device_id=(peer,), device_id_type=pl.DeviceIdType.MESH,
            )
        pl.semaphore_wait(barrier_sem, N_DEV - 1)

        rdmas = []
        for d in range(1, N_DEV):
            peer = lax.rem(my + d, N_DEV)
            rdma = pltpu.make_async_remote_copy(
                src_ref=x_ref.at[pl.ds(peer * m_per, m_per), :],
                dst_ref=comm_ref.at[d - 1],
                send_sem=send_sems.at[d - 1],
                recv_sem=recv_sems.at[d - 1],
                device_id=(peer,),
                device_id_type=pl.DeviceIdType.MESH,
            )
            rdma.start()
            rdmas.append(rdma)

        x_loc = x_ref[pl.ds(my * m_per, m_per), :]
        w_loc = w_ref[pl.ds(my * k_shard, k_shard), :]
        acc = jnp.dot(x_loc, w_loc, preferred_element_type=jnp.float32)

        for d in (1, 3, 2):
            rdmas[d - 1].wait_recv()
            src = lax.rem(my + N_DEV - d, N_DEV)
            w_blk = w_ref[pl.ds(src * k_shard, k_shard), :]
            acc = acc + jnp.dot(
                comm_ref[d - 1], w_blk, preferred_element_type=jnp.float32
            )

        for d in range(1, N_DEV):
            rdmas[d - 1].wait_send()

        out_ref[:, :] = acc * jax.nn.sigmoid(acc)

    return pl.pallas_call(
        body,
        out_shape=jax.ShapeDtypeStruct((m_per, n), jnp.float32),
        in_specs=[
            pl.BlockSpec(memory_space=pltpu.VMEM),
            pl.BlockSpec(memory_space=pltpu.VMEM),
        ],
        out_specs=pl.BlockSpec(memory_space=pltpu.VMEM),
        scratch_shapes=[
            pltpu.VMEM((N_DEV - 1, m_per, k_shard), jnp.float32),
            pltpu.SemaphoreType.DMA((N_DEV - 1,)),
            pltpu.SemaphoreType.DMA((N_DEV - 1,)),
        ],
        compiler_params=pltpu.CompilerParams(collective_id=0),
    )(x, w_mat)


# baseline (device time: 110637 ns/iter reference)
import jax
import jax.numpy as jnp
from jax import lax
from jax.experimental import pallas as pl
from jax.experimental.pallas import tpu as pltpu

N_DEV = 4


def kernel(x, w_mat):
    m_total, k_shard = x.shape
    k_total, n = w_mat.shape
    m_per = m_total // N_DEV

    def body(x_hbm, w_hbm, out_ref, comm_ref, x_loc, w_buf,
             send_sems, recv_sems, x_sem, w_sems):
        my = lax.axis_index("i")

        consume_d = (1, 3, 2)

        barrier_sem = pltpu.get_barrier_semaphore()
        for d in range(1, N_DEV):
            peer = lax.rem(my + d, N_DEV)
            pl.semaphore_signal(
                barrier_sem, inc=1,
                device_id=(peer,), device_id_type=pl.DeviceIdType.MESH,
            )
        pl.semaphore_wait(barrier_sem, N_DEV - 1)

        rdmas = []
        for d in range(1, N_DEV):
            peer = lax.rem(my + d, N_DEV)
            rdma = pltpu.make_async_remote_copy(
                src_ref=x_hbm.at[pl.ds(peer * m_per, m_per), :],
                dst_ref=comm_ref.at[d - 1],
                send_sem=send_sems.at[d - 1],
                recv_sem=recv_sems.at[d - 1],
                device_id=(peer,),
                device_id_type=pl.DeviceIdType.MESH,
            )
            rdma.start()
            rdmas.append(rdma)

        x_cp = pltpu.make_async_copy(
            x_hbm.at[pl.ds(my * m_per, m_per), :], x_loc, x_sem
        )
        x_cp.start()

        def w_block_copy(src_dev, slot):
            return pltpu.make_async_copy(
                w_hbm.at[pl.ds(src_dev * k_shard, k_shard), :],
                w_buf.at[slot],
                w_sems.at[slot],
            )

        w_cp = w_block_copy(my, 0)
        w_cp.start()
        src0 = lax.rem(my + N_DEV - consume_d[0], N_DEV)
        w_next = w_block_copy(src0, 1)
        w_next.start()

        x_cp.wait()
        w_cp.wait()
        out_ref[:, :] = jnp.dot(
            x_loc[:, :], w_buf[0], preferred_element_type=jnp.float32
        )

        for i, d in enumerate(consume_d):
            slot = (i + 1) % 2
            w_pending = w_next
            if i + 1 < len(consume_d):
                src_nxt = lax.rem(my + N_DEV - consume_d[i + 1], N_DEV)
                w_next = w_block_copy(src_nxt, i % 2)
            rdmas[d - 1].wait_recv()
            w_pending.wait()
            if i + 1 < len(consume_d):
                w_next.start()
            out_ref[:, :] = out_ref[:, :] + jnp.dot(
                comm_ref[d - 1], w_buf[slot],
                preferred_element_type=jnp.float32,
            )

        for d in range(1, N_DEV):
            rdmas[d - 1].wait_send()

        y = out_ref[:, :]
        out_ref[:, :] = y * jax.nn.sigmoid(y)

    return pl.pallas_call(
        body,
        out_shape=jax.ShapeDtypeStruct((m_per, n), jnp.float32),
        in_specs=[
            pl.BlockSpec(memory_space=pl.ANY),
            pl.BlockSpec(memory_space=pl.ANY),
        ],
        out_specs=pl.BlockSpec(memory_space=pltpu.VMEM),
        scratch_shapes=[
            pltpu.VMEM((N_DEV - 1, m_per, k_shard), jnp.float32),
            pltpu.VMEM((m_per, k_shard), jnp.float32),
            pltpu.VMEM((2, k_shard, n), jnp.float32),
            pltpu.SemaphoreType.DMA((N_DEV - 1,)),
            pltpu.SemaphoreType.DMA((N_DEV - 1,)),
            pltpu.SemaphoreType.DMA,
            pltpu.SemaphoreType.DMA((2,)),
        ],
        compiler_params=pltpu.CompilerParams(collective_id=0),
    )(x, w_mat)


# device time: 109276 ns/iter; 1.0125x vs baseline; 1.0125x over previous
import jax
import jax.numpy as jnp
from jax import lax
from jax.experimental import pallas as pl
from jax.experimental.pallas import tpu as pltpu

N_DEV = 4

_SEM_D1, _SEM_D3, _SEM_D2C0, _SEM_D2C1 = 0, 1, 2, 3
_SLOT = {1: 0, 2: 1, 3: 2}


def kernel(x, w_mat):
    m_total, k_shard = x.shape
    k_total, n = w_mat.shape
    m_per = m_total // N_DEV
    m_half = m_per // 2

    def body(x_hbm, w_hbm, out_ref, comm_ref, x_loc, w_buf,
             send_sems, recv_sems, x_sem, w_sems):
        my = lax.axis_index("i")

        barrier_sem = pltpu.get_barrier_semaphore()
        for d in range(1, N_DEV):
            peer = lax.rem(my + d, N_DEV)
            pl.semaphore_signal(
                barrier_sem, inc=1,
                device_id=(peer,), device_id_type=pl.DeviceIdType.MESH,
            )
        pl.semaphore_wait(barrier_sem, N_DEV - 1)

        def remote_copy(d, rows, sem_idx):
            peer = lax.rem(my + d, N_DEV)
            return pltpu.make_async_remote_copy(
                src_ref=x_hbm.at[pl.ds(peer * m_per + rows[0],
                                       rows[1] - rows[0]), :],
                dst_ref=comm_ref.at[_SLOT[d], pl.ds(rows[0],
                                                    rows[1] - rows[0]), :],
                send_sem=send_sems.at[sem_idx],
                recv_sem=recv_sems.at[sem_idx],
                device_id=(peer,),
                device_id_type=pl.DeviceIdType.MESH,
            )

        rdma_d1 = remote_copy(1, (0, m_per), _SEM_D1)
        rdma_d3 = remote_copy(3, (0, m_per), _SEM_D3)
        rdma_d1.start()
        rdma_d3.start()

        x_cp = pltpu.make_async_copy(
            x_hbm.at[pl.ds(my * m_per, m_per), :], x_loc, x_sem
        )
        x_cp.start()

        def w_block_copy(src_dev, slot):
            return pltpu.make_async_copy(
                w_hbm.at[pl.ds(src_dev * k_shard, k_shard), :],
                w_buf.at[slot],
                w_sems.at[slot],
            )

        w_cp0 = w_block_copy(my, 0)
        w_cp1 = w_block_copy(lax.rem(my + N_DEV - 1, N_DEV), 1)
        w_cp0.start()
        w_cp1.start()

        def set_strip(row0, src_block, w_slot):
            out_ref[pl.ds(row0, m_half), :] = jnp.dot(
                src_block, w_buf[w_slot], preferred_element_type=jnp.float32
            )

        def acc_strip(row0, src_block, w_slot):
            out_ref[pl.ds(row0, m_half), :] = out_ref[
                pl.ds(row0, m_half), :
            ] + jnp.dot(
                src_block, w_buf[w_slot], preferred_element_type=jnp.float32
            )

        def silu_strip(row0):
            y = out_ref[pl.ds(row0, m_half), :]
            out_ref[pl.ds(row0, m_half), :] = y * jax.nn.sigmoid(y)

        x_cp.wait()
        w_cp0.wait()
        for r in (0, m_half):
            set_strip(r, x_loc[pl.ds(r, m_half), :], 0)

        rdma_d1.wait_send()
        rdma_d3.wait_send()
        rdma_d2c0 = remote_copy(2, (0, m_half), _SEM_D2C0)
        rdma_d2c1 = remote_copy(2, (m_half, m_per), _SEM_D2C1)
        rdma_d2c0.start()
        rdma_d2c1.start()

        rdma_d1.wait_recv()
        w_cp1.wait()
        w_cp0 = w_block_copy(lax.rem(my + 1, N_DEV), 0)
        w_cp0.start()
        for r in (0, m_half):
            acc_strip(r, comm_ref[_SLOT[1], pl.ds(r, m_half), :], 1)

        rdma_d3.wait_recv()
        w_cp0.wait()
        w_cp1 = w_block_copy(lax.rem(my + 2, N_DEV), 1)
        w_cp1.start()
        for r in (0, m_half):
            acc_strip(r, comm_ref[_SLOT[3], pl.ds(r, m_half), :], 0)

        rdma_d2c0.wait_recv()
        w_cp1.wait()
        acc_strip(0, comm_ref[_SLOT[2], pl.ds(0, m_half), :], 1)
        silu_strip(0)

        rdma_d2c1.wait_recv()
        acc_strip(m_half, comm_ref[_SLOT[2], pl.ds(m_half, m_half), :], 1)
        silu_strip(m_half)

        rdma_d2c0.wait_send()
        rdma_d2c1.wait_send()

    return pl.pallas_call(
        body,
        out_shape=jax.ShapeDtypeStruct((m_per, n), jnp.float32),
        in_specs=[
            pl.BlockSpec(memory_space=pl.ANY),
            pl.BlockSpec(memory_space=pl.ANY),
        ],
        out_specs=pl.BlockSpec(memory_space=pltpu.VMEM),
        scratch_shapes=[
            pltpu.VMEM((N_DEV - 1, m_per, k_shard), jnp.float32),
            pltpu.VMEM((m_per, k_shard), jnp.float32),
            pltpu.VMEM((2, k_shard, n), jnp.float32),
            pltpu.SemaphoreType.DMA((4,)),
            pltpu.SemaphoreType.DMA((4,)),
            pltpu.SemaphoreType.DMA,
            pltpu.SemaphoreType.DMA((2,)),
        ],
        compiler_params=pltpu.CompilerParams(
            collective_id=0,
            vmem_limit_bytes=60 * 1024 * 1024,
        ),
    )(x, w_mat)


# device time: 107663 ns/iter; 1.0276x vs baseline; 1.0150x over previous
import jax
import jax.numpy as jnp
from jax import lax
from jax.experimental import pallas as pl
from jax.experimental.pallas import tpu as pltpu

N_DEV = 4

_SEM_D1, _SEM_D3, _SEM_D2 = 0, 1, 2
_SLOT = {1: 0, 2: 1, 3: 2}
_N_DIAG_CHUNKS = 4


def kernel(x, w_mat):
    m_total, k_shard = x.shape
    k_total, n = w_mat.shape
    m_per = m_total // N_DEV
    m_half = m_per // 2
    m_chunk = m_per // _N_DIAG_CHUNKS

    def body(x_hbm, w_hbm, out_ref, comm_ref, x_loc, w_buf,
             send_sems, recv_sems, x_sem, w_sems):
        my = lax.axis_index("i")

        barrier_sem = pltpu.get_barrier_semaphore()
        for d in range(1, N_DEV):
            peer = lax.rem(my + d, N_DEV)
            pl.semaphore_signal(
                barrier_sem, inc=1,
                device_id=(peer,), device_id_type=pl.DeviceIdType.MESH,
            )
        pl.semaphore_wait(barrier_sem, 2)

        def remote_copy(d, rows, sem_idx):
            peer = lax.rem(my + d, N_DEV)
            return pltpu.make_async_remote_copy(
                src_ref=x_hbm.at[pl.ds(peer * m_per + rows[0],
                                       rows[1] - rows[0]), :],
                dst_ref=comm_ref.at[_SLOT[d], pl.ds(rows[0],
                                                    rows[1] - rows[0]), :],
                send_sem=send_sems.at[sem_idx],
                recv_sem=recv_sems.at[sem_idx],
                device_id=(peer,),
                device_id_type=pl.DeviceIdType.MESH,
            )

        rdma_d1 = remote_copy(1, (0, m_per), _SEM_D1)
        rdma_d3 = remote_copy(3, (0, m_per), _SEM_D3)
        rdma_d1.start()
        rdma_d3.start()

        x_cp = pltpu.make_async_copy(
            x_hbm.at[pl.ds(my * m_per, m_per), :], x_loc, x_sem
        )
        x_cp.start()

        def w_block_copy(src_dev, slot):
            return pltpu.make_async_copy(
                w_hbm.at[pl.ds(src_dev * k_shard, k_shard), :],
                w_buf.at[slot],
                w_sems.at[slot],
            )

        w_cp0 = w_block_copy(my, 0)
        w_cp1 = w_block_copy(lax.rem(my + N_DEV - 1, N_DEV), 1)
        w_cp0.start()
        w_cp1.start()

        def set_strip(row0, src_block, w_slot):
            out_ref[pl.ds(row0, m_half), :] = jnp.dot(
                src_block, w_buf[w_slot], preferred_element_type=jnp.float32
            )

        def acc_strip(row0, rows, src_block, w_slot):
            out_ref[pl.ds(row0, rows), :] = out_ref[
                pl.ds(row0, rows), :
            ] + jnp.dot(
                src_block, w_buf[w_slot], preferred_element_type=jnp.float32
            )

        def silu_strip(row0, rows):
            y = out_ref[pl.ds(row0, rows), :]
            out_ref[pl.ds(row0, rows), :] = y * jax.nn.sigmoid(y)

        x_cp.wait()
        w_cp0.wait()
        for r in (0, m_half):
            set_strip(r, x_loc[pl.ds(r, m_half), :], 0)

        rdma_d1.wait_send()
        rdma_d3.wait_send()
        pl.semaphore_wait(barrier_sem, 1)
        rdma_d2 = []
        for c in range(_N_DIAG_CHUNKS):
            r = pltpu.make_async_remote_copy(
                src_ref=x_hbm.at[
                    pl.ds(lax.rem(my + 2, N_DEV) * m_per + c * m_chunk,
                          m_chunk), :],
                dst_ref=comm_ref.at[_SLOT[2], pl.ds(c * m_chunk, m_chunk), :],
                send_sem=send_sems.at[_SEM_D2 + c],
                recv_sem=recv_sems.at[_SEM_D2 + c],
                device_id=(lax.rem(my + 2, N_DEV),),
                device_id_type=pl.DeviceIdType.MESH,
            )
            r.start()
            rdma_d2.append(r)

        rdma_d1.wait_recv()
        w_cp1.wait()
        w_cp0 = w_block_copy(lax.rem(my + 1, N_DEV), 0)
        w_cp0.start()
        for r in (0, m_half):
            acc_strip(r, m_half, comm_ref[_SLOT[1], pl.ds(r, m_half), :], 1)

        rdma_d3.wait_recv()
        w_cp0.wait()
        w_cp1 = w_block_copy(lax.rem(my + 2, N_DEV), 1)
        w_cp1.start()
        for r in (0, m_half):
            acc_strip(r, m_half, comm_ref[_SLOT[3], pl.ds(r, m_half), :], 0)

        w_cp1.wait()
        for c in range(_N_DIAG_CHUNKS):
            rdma_d2[c].wait_recv()
            r0 = c * m_chunk
            acc_strip(r0, m_chunk,
                      comm_ref[_SLOT[2], pl.ds(r0, m_chunk), :], 1)
            silu_strip(r0, m_chunk)

        for c in range(_N_DIAG_CHUNKS):
            rdma_d2[c].wait_send()

    return pl.pallas_call(
        body,
        out_shape=jax.ShapeDtypeStruct((m_per, n), jnp.float32),
        in_specs=[
            pl.BlockSpec(memory_space=pl.ANY),
            pl.BlockSpec(memory_space=pl.ANY),
        ],
        out_specs=pl.BlockSpec(memory_space=pltpu.VMEM),
        scratch_shapes=[
            pltpu.VMEM((N_DEV - 1, m_per, k_shard), jnp.float32),
            pltpu.VMEM((m_per, k_shard), jnp.float32),
            pltpu.VMEM((2, k_shard, n), jnp.float32),
            pltpu.SemaphoreType.DMA((2 + _N_DIAG_CHUNKS,)),
            pltpu.SemaphoreType.DMA((2 + _N_DIAG_CHUNKS,)),
            pltpu.SemaphoreType.DMA,
            pltpu.SemaphoreType.DMA((2,)),
        ],
        compiler_params=pltpu.CompilerParams(
            collective_id=0,
            vmem_limit_bytes=60 * 1024 * 1024,
        ),
    )(x, w_mat)


# device time: 67996 ns/iter; 1.6271x vs baseline; 1.5834x over previous
import jax
import jax.numpy as jnp
from jax import lax
from jax.experimental import pallas as pl
from jax.experimental.pallas import tpu as pltpu

N_DEV = 4

_SEM_D1, _SEM_D3, _SEM_D2 = 0, 1, 2
_SLOT = {1: 0, 2: 1, 3: 2}
_N_DIAG_CHUNKS = 4


def kernel(x, w_mat):
    m_total, k_shard = x.shape
    k_total, n = w_mat.shape
    m_per = m_total // N_DEV
    m_half = m_per // 2
    m_chunk = m_per // _N_DIAG_CHUNKS

    def body(x_hbm, w_hbm, out_ref, comm_ref, send_buf, x_stage, x_loc,
             w_buf, send_sems, recv_sems, stage_sems, x_sem, w_sems):
        my = lax.axis_index("i")

        barrier_sem = pltpu.get_barrier_semaphore()
        for d in range(1, N_DEV):
            peer = lax.rem(my + d, N_DEV)
            pl.semaphore_signal(
                barrier_sem, inc=1,
                device_id=(peer,), device_id_type=pl.DeviceIdType.MESH,
            )
        pl.semaphore_wait(barrier_sem, 2)

        def stage_copy(d, stage_slot):
            peer = lax.rem(my + d, N_DEV)
            return pltpu.make_async_copy(
                x_hbm.at[pl.ds(peer * m_per, m_per), :],
                x_stage.at[stage_slot],
                stage_sems.at[stage_slot],
            )

        def neighbor_rdma(d, sem_idx):
            peer = lax.rem(my + d, N_DEV)
            return pltpu.make_async_remote_copy(
                src_ref=send_buf.at[_SLOT[d]],
                dst_ref=comm_ref.at[_SLOT[d]],
                send_sem=send_sems.at[sem_idx],
                recv_sem=recv_sems.at[sem_idx],
                device_id=(peer,),
                device_id_type=pl.DeviceIdType.MESH,
            )

        cp1 = stage_copy(1, 0)
        cp3 = stage_copy(3, 1)
        cp1.start()
        cp3.start()

        x_cp = pltpu.make_async_copy(
            x_hbm.at[pl.ds(my * m_per, m_per), :], x_loc, x_sem
        )
        x_cp.start()

        def w_block_copy(src_dev, slot):
            return pltpu.make_async_copy(
                w_hbm.at[pl.ds(src_dev * k_shard, k_shard), :],
                w_buf.at[slot],
                w_sems.at[slot],
            )

        w_cp0 = w_block_copy(my, 0)
        w_cp1 = w_block_copy(lax.rem(my + N_DEV - 1, N_DEV), 1)
        w_cp0.start()
        w_cp1.start()

        cp1.wait()
        send_buf[_SLOT[1]] = x_stage[0].astype(jnp.bfloat16)
        rdma_d1 = neighbor_rdma(1, _SEM_D1)
        rdma_d1.start()

        cp3.wait()
        send_buf[_SLOT[3]] = x_stage[1].astype(jnp.bfloat16)
        rdma_d3 = neighbor_rdma(3, _SEM_D3)
        rdma_d3.start()

        cp_diag = stage_copy(2, 0)
        cp_diag.start()
        cp_diag.wait()
        send_buf[_SLOT[2]] = x_stage[0].astype(jnp.bfloat16)

        def set_strip(row0, src_block, w_slot):
            out_ref[pl.ds(row0, m_half), :] = jnp.dot(
                src_block, w_buf[w_slot], preferred_element_type=jnp.float32
            )

        def acc_strip(row0, rows, src_block, w_slot):
            out_ref[pl.ds(row0, rows), :] = out_ref[
                pl.ds(row0, rows), :
            ] + jnp.dot(
                src_block.astype(jnp.float32), w_buf[w_slot],
                preferred_element_type=jnp.float32,
            )

        def silu_strip(row0, rows):
            y = out_ref[pl.ds(row0, rows), :]
            out_ref[pl.ds(row0, rows), :] = y * jax.nn.sigmoid(y)

        x_cp.wait()
        w_cp0.wait()
        for r in (0, m_half):
            set_strip(r, x_loc[pl.ds(r, m_half), :], 0)

        rdma_d1.wait_send()
        rdma_d3.wait_send()
        pl.semaphore_wait(barrier_sem, 1)
        rdma_d2 = []
        for c in range(_N_DIAG_CHUNKS):
            r = pltpu.make_async_remote_copy(
                src_ref=send_buf.at[_SLOT[2], pl.ds(c * m_chunk, m_chunk), :],
                dst_ref=comm_ref.at[_SLOT[2], pl.ds(c * m_chunk, m_chunk), :],
                send_sem=send_sems.at[_SEM_D2 + c],
                recv_sem=recv_sems.at[_SEM_D2 + c],
                device_id=(lax.rem(my + 2, N_DEV),),
                device_id_type=pl.DeviceIdType.MESH,
            )
            r.start()
            rdma_d2.append(r)

        rdma_d1.wait_recv()
        w_cp1.wait()
        w_cp0 = w_block_copy(lax.rem(my + 1, N_DEV), 0)
        w_cp0.start()
        for r in (0, m_half):
            acc_strip(r, m_half, comm_ref[_SLOT[1], pl.ds(r, m_half), :], 1)

        rdma_d3.wait_recv()
        w_cp0.wait()
        w_cp1 = w_block_copy(lax.rem(my + 2, N_DEV), 1)
        w_cp1.start()
        for r in (0, m_half):
            acc_strip(r, m_half, comm_ref[_SLOT[3], pl.ds(r, m_half), :], 0)

        w_cp1.wait()
        for c in range(_N_DIAG_CHUNKS):
            rdma_d2[c].wait_recv()
            r0 = c * m_chunk
            acc_strip(r0, m_chunk,
                      comm_ref[_SLOT[2], pl.ds(r0, m_chunk), :], 1)
            silu_strip(r0, m_chunk)

        for c in range(_N_DIAG_CHUNKS):
            rdma_d2[c].wait_send()

    return pl.pallas_call(
        body,
        out_shape=jax.ShapeDtypeStruct((m_per, n), jnp.float32),
        in_specs=[
            pl.BlockSpec(memory_space=pl.ANY),
            pl.BlockSpec(memory_space=pl.ANY),
        ],
        out_specs=pl.BlockSpec(memory_space=pltpu.VMEM),
        scratch_shapes=[
            pltpu.VMEM((N_DEV - 1, m_per, k_shard), jnp.bfloat16),
            pltpu.VMEM((N_DEV - 1, m_per, k_shard), jnp.bfloat16),
            pltpu.VMEM((2, m_per, k_shard), jnp.float32),
            pltpu.VMEM((m_per, k_shard), jnp.float32),
            pltpu.VMEM((2, k_shard, n), jnp.float32),
            pltpu.SemaphoreType.DMA((2 + _N_DIAG_CHUNKS,)),
            pltpu.SemaphoreType.DMA((2 + _N_DIAG_CHUNKS,)),
            pltpu.SemaphoreType.DMA((2,)),
            pltpu.SemaphoreType.DMA,
            pltpu.SemaphoreType.DMA((2,)),
        ],
        compiler_params=pltpu.CompilerParams(
            collective_id=0,
            vmem_limit_bytes=60 * 1024 * 1024,
        ),
    )(x, w_mat)


# device time: 48677 ns/iter; 2.2729x vs baseline; 1.3969x over previous
import jax
import jax.numpy as jnp
from jax import lax
from jax.experimental import pallas as pl
from jax.experimental.pallas import tpu as pltpu

N_DEV = 4

_SEM_D1, _SEM_D3, _SEM_D2 = 0, 1, 2
_SLOT = {1: 0, 2: 1, 3: 2}
_N_DIAG_CHUNKS = 4


def kernel(x, w_mat):
    m_total, k_shard = x.shape
    k_total, n = w_mat.shape
    m_per = m_total // N_DEV
    m_half = m_per // 2
    m_chunk = m_per // _N_DIAG_CHUNKS

    def body(x_hbm, w_hbm, out_ref, comm_ref, send_buf, x_stage, x_loc,
             w_buf, scale_snd, scale_rcv, send_sems, recv_sems,
             sc_send_sems, sc_recv_sems, stage_sems, x_sem, w_sems):
        my = lax.axis_index("i")

        barrier_sem = pltpu.get_barrier_semaphore()
        for d in range(1, N_DEV):
            peer = lax.rem(my + d, N_DEV)
            pl.semaphore_signal(
                barrier_sem, inc=1,
                device_id=(peer,), device_id_type=pl.DeviceIdType.MESH,
            )
        pl.semaphore_wait(barrier_sem, 2)

        def stage_copy(d, stage_slot):
            peer = lax.rem(my + d, N_DEV)
            return pltpu.make_async_copy(
                x_hbm.at[pl.ds(peer * m_per, m_per), :],
                x_stage.at[stage_slot],
                stage_sems.at[stage_slot],
            )

        def quantize(stage_slot, out_slot):
            blk = x_stage[stage_slot]
            mx = jnp.maximum(jnp.max(jnp.abs(blk)), 1e-30)
            send_buf[out_slot] = jnp.round(blk * (127.0 / mx)).astype(
                jnp.int8
            )
            scale_snd[out_slot] = jnp.full((8, 128), mx / 127.0, jnp.float32)

        def scale_rdma(d, slot):
            peer = lax.rem(my + d, N_DEV)
            return pltpu.make_async_remote_copy(
                src_ref=scale_snd.at[slot],
                dst_ref=scale_rcv.at[slot],
                send_sem=sc_send_sems.at[slot],
                recv_sem=sc_recv_sems.at[slot],
                device_id=(peer,),
                device_id_type=pl.DeviceIdType.MESH,
            )

        def neighbor_rdma(d, sem_idx):
            peer = lax.rem(my + d, N_DEV)
            return pltpu.make_async_remote_copy(
                src_ref=send_buf.at[_SLOT[d]],
                dst_ref=comm_ref.at[_SLOT[d]],
                send_sem=send_sems.at[sem_idx],
                recv_sem=recv_sems.at[sem_idx],
                device_id=(peer,),
                device_id_type=pl.DeviceIdType.MESH,
            )

        cp1 = stage_copy(1, 0)
        cp3 = stage_copy(3, 1)
        cp1.start()
        cp3.start()

        x_cp = pltpu.make_async_copy(
            x_hbm.at[pl.ds(my * m_per, m_per), :], x_loc, x_sem
        )
        x_cp.start()

        def w_block_copy(src_dev, slot):
            return pltpu.make_async_copy(
                w_hbm.at[pl.ds(src_dev * k_shard, k_shard), :],
                w_buf.at[slot],
                w_sems.at[slot],
            )

        w_cp0 = w_block_copy(my, 0)
        w_cp1 = w_block_copy(lax.rem(my + N_DEV - 1, N_DEV), 1)
        w_cp0.start()
        w_cp1.start()

        cp1.wait()
        quantize(0, _SLOT[1])
        sc1 = scale_rdma(1, _SLOT[1])
        sc1.start()
        rdma_d1 = neighbor_rdma(1, _SEM_D1)
        rdma_d1.start()

        cp3.wait()
        quantize(1, _SLOT[3])
        sc3 = scale_rdma(3, _SLOT[3])
        sc3.start()
        rdma_d3 = neighbor_rdma(3, _SEM_D3)
        rdma_d3.start()

        cp_diag = stage_copy(2, 0)
        cp_diag.start()
        cp_diag.wait()
        quantize(0, _SLOT[2])

        def set_strip(row0, src_block, w_slot):
            out_ref[pl.ds(row0, m_half), :] = jnp.dot(
                src_block, w_buf[w_slot], preferred_element_type=jnp.float32
            )

        def acc_strip(row0, rows, src_block, w_slot, scale_slot):
            out_ref[pl.ds(row0, rows), :] = out_ref[
                pl.ds(row0, rows), :
            ] + jnp.dot(
                src_block.astype(jnp.float32), w_buf[w_slot],
                preferred_element_type=jnp.float32,
            ) * scale_rcv[scale_slot, 0:1, 0:1]

        def silu_strip(row0, rows):
            y = out_ref[pl.ds(row0, rows), :]
            out_ref[pl.ds(row0, rows), :] = y * jax.nn.sigmoid(y)

        x_cp.wait()
        w_cp0.wait()
        for r in (0, m_half):
            set_strip(r, x_loc[pl.ds(r, m_half), :], 0)

        rdma_d1.wait_send()
        rdma_d3.wait_send()
        pl.semaphore_wait(barrier_sem, 1)
        sc2 = scale_rdma(2, _SLOT[2])
        sc2.start()
        rdma_d2 = []
        for c in range(_N_DIAG_CHUNKS):
            r = pltpu.make_async_remote_copy(
                src_ref=send_buf.at[_SLOT[2], pl.ds(c * m_chunk, m_chunk), :],
                dst_ref=comm_ref.at[_SLOT[2], pl.ds(c * m_chunk, m_chunk), :],
                send_sem=send_sems.at[_SEM_D2 + c],
                recv_sem=recv_sems.at[_SEM_D2 + c],
                device_id=(lax.rem(my + 2, N_DEV),),
                device_id_type=pl.DeviceIdType.MESH,
            )
            r.start()
            rdma_d2.append(r)

        sc1.wait_recv()
        rdma_d1.wait_recv()
        w_cp1.wait()
        w_cp0 = w_block_copy(lax.rem(my + 1, N_DEV), 0)
        w_cp0.start()
        for r in (0, m_half):
            acc_strip(r, m_half, comm_ref[_SLOT[1], pl.ds(r, m_half), :],
                      1, _SLOT[1])

        sc3.wait_recv()
        rdma_d3.wait_recv()
        w_cp0.wait()
        w_cp1 = w_block_copy(lax.rem(my + 2, N_DEV), 1)
        w_cp1.start()
        for r in (0, m_half):
            acc_strip(r, m_half, comm_ref[_SLOT[3], pl.ds(r, m_half), :],
                      0, _SLOT[3])

        w_cp1.wait()
        sc2.wait_recv()
        for c in range(_N_DIAG_CHUNKS):
            rdma_d2[c].wait_recv()
            r0 = c * m_chunk
            acc_strip(r0, m_chunk,
                      comm_ref[_SLOT[2], pl.ds(r0, m_chunk), :],
                      1, _SLOT[2])
            silu_strip(r0, m_chunk)

        for c in range(_N_DIAG_CHUNKS):
            rdma_d2[c].wait_send()
        sc1.wait_send()
        sc3.wait_send()
        sc2.wait_send()

    return pl.pallas_call(
        body,
        out_shape=jax.ShapeDtypeStruct((m_per, n), jnp.float32),
        in_specs=[
            pl.BlockSpec(memory_space=pl.ANY),
            pl.BlockSpec(memory_space=pl.ANY),
        ],
        out_specs=pl.BlockSpec(memory_space=pltpu.VMEM),
        scratch_shapes=[
            pltpu.VMEM((N_DEV - 1, m_per, k_shard), jnp.int8),
            pltpu.VMEM((N_DEV - 1, m_per, k_shard), jnp.int8),
            pltpu.VMEM((2, m_per, k_shard), jnp.float32),
            pltpu.VMEM((m_per, k_shard), jnp.float32),
            pltpu.VMEM((2, k_shard, n), jnp.float32),
            pltpu.VMEM((N_DEV - 1, 8, 128), jnp.float32),
            pltpu.VMEM((N_DEV - 1, 8, 128), jnp.float32),
            pltpu.SemaphoreType.DMA((2 + _N_DIAG_CHUNKS,)),
            pltpu.SemaphoreType.DMA((2 + _N_DIAG_CHUNKS,)),
            pltpu.SemaphoreType.DMA((N_DEV - 1,)),
            pltpu.SemaphoreType.DMA((N_DEV - 1,)),
            pltpu.SemaphoreType.DMA((2,)),
            pltpu.SemaphoreType.DMA,
            pltpu.SemaphoreType.DMA((2,)),
        ],
        compiler_params=pltpu.CompilerParams(
            collective_id=0,
            vmem_limit_bytes=60 * 1024 * 1024,
        ),
    )(x, w_mat)


# device time: 44641 ns/iter; 2.4784x vs baseline; 1.0904x over previous
import jax
import jax.numpy as jnp
from jax import lax
from jax.experimental import pallas as pl
from jax.experimental.pallas import tpu as pltpu

N_DEV = 4

_SLOT = {1: 0, 2: 1, 3: 2}
_N_DIAG_CHUNKS = 4
_N_DATA_SEMS = 4 + _N_DIAG_CHUNKS
_N_SCALES = 5


def kernel(x, w_mat):
    m_total, k_shard = x.shape
    k_total, n = w_mat.shape
    m_per = m_total // N_DEV
    m_half = m_per // 2
    m_chunk = m_per // _N_DIAG_CHUNKS

    def body(x_hbm, w_hbm, out_ref, comm_ref, send_buf, x_stage, x_loc,
             w_buf, scale_snd, scale_rcv, send_sems, recv_sems,
             sc_send_sems, sc_recv_sems, stage_sems, x_sem, w_sems):
        my = lax.axis_index("i")
        peer = {d: lax.rem(my + d, N_DEV) for d in (1, 2, 3)}

        def stage_half(d, stage_slot, h, sem_idx):
            return pltpu.make_async_copy(
                x_hbm.at[pl.ds(peer[d] * m_per + h * m_half, m_half), :],
                x_stage.at[stage_slot, pl.ds(h * m_half, m_half), :],
                stage_sems.at[sem_idx],
            )

        cps = {
            (1, 0): stage_half(1, 0, 0, 0),
            (1, 1): stage_half(1, 0, 1, 1),
            (3, 0): stage_half(3, 1, 0, 2),
            (3, 1): stage_half(3, 1, 1, 3),
        }
        for cp in cps.values():
            cp.start()

        x_cp = pltpu.make_async_copy(
            x_hbm.at[pl.ds(my * m_per, m_per), :], x_loc, x_sem
        )
        x_cp.start()

        def w_block_copy(src_dev, slot):
            return pltpu.make_async_copy(
                w_hbm.at[pl.ds(src_dev * k_shard, k_shard), :],
                w_buf.at[slot],
                w_sems.at[slot],
            )

        w_cp0 = w_block_copy(my, 0)
        w_cp1 = w_block_copy(lax.rem(my + N_DEV - 1, N_DEV), 1)
        w_cp0.start()
        w_cp1.start()

        barrier_sem = pltpu.get_barrier_semaphore()
        for d in (1, 2, 3):
            pl.semaphore_signal(
                barrier_sem, inc=1,
                device_id=(peer[d],), device_id_type=pl.DeviceIdType.MESH,
            )
        pl.semaphore_wait(barrier_sem, 2)

        def quantize(src_block, slot, row0, rows, scale_slot):
            mx = jnp.maximum(jnp.max(jnp.abs(src_block)), 1e-30)
            send_buf[slot, pl.ds(row0, rows), :] = jnp.round(
                src_block * (127.0 / mx)
            ).astype(jnp.int8)
            scale_snd[scale_slot] = jnp.full((8, 128), mx / 127.0,
                                             jnp.float32)

        def data_rdma(d, row0, rows, sem_idx):
            return pltpu.make_async_remote_copy(
                src_ref=send_buf.at[_SLOT[d], pl.ds(row0, rows), :],
                dst_ref=comm_ref.at[_SLOT[d], pl.ds(row0, rows), :],
                send_sem=send_sems.at[sem_idx],
                recv_sem=recv_sems.at[sem_idx],
                device_id=(peer[d],),
                device_id_type=pl.DeviceIdType.MESH,
            )

        def scale_rdma(d, scale_slot):
            return pltpu.make_async_remote_copy(
                src_ref=scale_snd.at[scale_slot],
                dst_ref=scale_rcv.at[scale_slot],
                send_sem=sc_send_sems.at[scale_slot],
                recv_sem=sc_recv_sems.at[scale_slot],
                device_id=(peer[d],),
                device_id_type=pl.DeviceIdType.MESH,
            )

        phase_a = []
        for d, stage_slot, sem_base in ((1, 0, 0), (3, 1, 2)):
            for h in (0, 1):
                cps[(d, h)].wait()
                quantize(
                    x_stage[stage_slot, pl.ds(h * m_half, m_half), :],
                    _SLOT[d], h * m_half, m_half, sem_base + h,
                )
                sc = scale_rdma(d, sem_base + h)
                sc.start()
                da = data_rdma(d, h * m_half, m_half, sem_base + h)
                da.start()
                phase_a.append((sc, da))

        cp_diag = pltpu.make_async_copy(
            x_hbm.at[pl.ds(peer[2] * m_per, m_per), :],
            x_stage.at[0],
            stage_sems.at[0],
        )
        cp_diag.start()
        cp_diag.wait()
        quantize(x_stage[0], _SLOT[2], 0, m_per, 4)

        def set_strip(row0, src_block, w_slot):
            out_ref[pl.ds(row0, m_half), :] = jnp.dot(
                src_block, w_buf[w_slot], preferred_element_type=jnp.float32
            )

        def acc_strip(row0, rows, src_block, w_slot, scale_slot):
            out_ref[pl.ds(row0, rows), :] = out_ref[
                pl.ds(row0, rows), :
            ] + jnp.dot(
                src_block.astype(jnp.float32), w_buf[w_slot],
                preferred_element_type=jnp.float32,
            ) * scale_rcv[scale_slot, 0:1, 0:1]

        def silu_strip(row0, rows):
            y = out_ref[pl.ds(row0, rows), :]
            out_ref[pl.ds(row0, rows), :] = y * jax.nn.sigmoid(y)

        x_cp.wait()
        w_cp0.wait()
        for r in (0, m_half):
            set_strip(r, x_loc[pl.ds(r, m_half), :], 0)
        w_cp0 = w_block_copy(lax.rem(my + 1, N_DEV), 0)
        w_cp0.start()

        for sc, da in phase_a:
            sc.wait_send()
            da.wait_send()
        pl.semaphore_wait(barrier_sem, 1)
        sc_diag = scale_rdma(2, 4)
        sc_diag.start()
        rdma_d2 = []
        for c in range(_N_DIAG_CHUNKS):
            r = data_rdma(2, c * m_chunk, m_chunk, 4 + c)
            r.start()
            rdma_d2.append(r)

        for h in (0, 1):
            phase_a[h][0].wait_recv()
            phase_a[h][1].wait_recv()
            if h == 0:
                w_cp1.wait()
            acc_strip(h * m_half, m_half,
                      comm_ref[_SLOT[1], pl.ds(h * m_half, m_half), :],
                      1, h)
        w_cp1 = w_block_copy(lax.rem(my + 2, N_DEV), 1)
        w_cp1.start()

        for h in (0, 1):
            phase_a[2 + h][0].wait_recv()
            phase_a[2 + h][1].wait_recv()
            if h == 0:
                w_cp0.wait()
            acc_strip(h * m_half, m_half,
                      comm_ref[_SLOT[3], pl.ds(h * m_half, m_half), :],
                      0, 2 + h)

        w_cp1.wait()
        sc_diag.wait_recv()
        for c in range(_N_DIAG_CHUNKS):
            rdma_d2[c].wait_recv()
            r0 = c * m_chunk
            acc_strip(r0, m_chunk,
                      comm_ref[_SLOT[2], pl.ds(r0, m_chunk), :], 1, 4)
            silu_strip(r0, m_chunk)

        for c in range(_N_DIAG_CHUNKS):
            rdma_d2[c].wait_send()
        sc_diag.wait_send()

    return pl.pallas_call(
        body,
        out_shape=jax.ShapeDtypeStruct((m_per, n), jnp.float32),
        in_specs=[
            pl.BlockSpec(memory_space=pl.ANY),
            pl.BlockSpec(memory_space=pl.ANY),
        ],
        out_specs=pl.BlockSpec(memory_space=pltpu.VMEM),
        scratch_shapes=[
            pltpu.VMEM((N_DEV - 1, m_per, k_shard), jnp.int8),
            pltpu.VMEM((N_DEV - 1, m_per, k_shard), jnp.int8),
            pltpu.VMEM((2, m_per, k_shard), jnp.float32),
            pltpu.VMEM((m_per, k_shard), jnp.float32),
            pltpu.VMEM((2, k_shard, n), jnp.float32),
            pltpu.VMEM((_N_SCALES, 8, 128), jnp.float32),
            pltpu.VMEM((_N_SCALES, 8, 128), jnp.float32),
            pltpu.SemaphoreType.DMA((_N_DATA_SEMS,)),
            pltpu.SemaphoreType.DMA((_N_DATA_SEMS,)),
            pltpu.SemaphoreType.DMA((_N_SCALES,)),
            pltpu.SemaphoreType.DMA((_N_SCALES,)),
            pltpu.SemaphoreType.DMA((4,)),
            pltpu.SemaphoreType.DMA,
            pltpu.SemaphoreType.DMA((2,)),
        ],
        compiler_params=pltpu.CompilerParams(
            collective_id=0,
            vmem_limit_bytes=60 * 1024 * 1024,
        ),
    )(x, w_mat)
